# baseline (speedup 1.0000x reference)
import functools
import math

import jax
import jax.numpy as jnp
from jax import lax
from jax.experimental import pallas as pl
from jax.experimental.pallas import tpu as pltpu
from jax.experimental.pallas import tpu_sc as plsc

NC = 2
NS = 16
NW = NC * NS
CB = 128

BN_SCALE = 1.0 / math.sqrt(1.0 + 1e-5)


NB = 8
NG = 6


def _make_sc_agg(n_pad, ch, dim):
    rpt = n_pad // NS
    mesh = plsc.VectorSubcoreMesh(core_axis_name="c", subcore_axis_name="s")

    @functools.partial(
        pl.kernel,
        out_type=jax.ShapeDtypeStruct((NC, n_pad, dim), jnp.float32),
        mesh=mesh,
        scratch_types=[
            pltpu.VMEM((ch, CB), jnp.int32),
            pltpu.VMEM((ch, CB), jnp.int32),
            [pltpu.VMEM((CB, dim), jnp.float32) for _ in range(NB)],
            [pltpu.SemaphoreType.DMA for _ in range(NB)],
            [pltpu.SemaphoreType.DMA for _ in range(NB)],
            pltpu.VMEM_SHARED((n_pad, dim), jnp.float32),
        ],
        compiler_params=pltpu.CompilerParams(use_tc_tiling_on_sc=False),
    )
    def agg(t_hbm, srcm, dstm, zeros_hbm, out_hbm, src_v, dst_v, rows, gsems,
            ssems, acc):
        cid = lax.axis_index("c")
        sid = lax.axis_index("s")
        wid = sid * NC + cid
        pltpu.sync_copy(srcm.at[wid], src_v)
        pltpu.sync_copy(dstm.at[wid], dst_v)
        sl = pl.ds(sid * rpt, rpt)
        pltpu.sync_copy(zeros_hbm.at[sl], acc.at[sl])
        plsc.subcore_barrier()

        for b in range(NG):
            pltpu.async_copy(t_hbm.at[src_v.at[b]], rows[b], gsems[b])

        def outer(g, carry):
            for b in range(NB):
                j = g * NB + b
                pltpu.make_async_copy(t_hbm.at[src_v.at[b]], rows[b],
                                      gsems[b]).wait()
                pltpu.async_copy(rows[b], acc.at[dst_v.at[j]], ssems[b],
                                 add=True)
                b2 = (b + NG) % NB
                jg = j + NG

                @pl.when(jg < ch)
                def _():
                    @pl.when(jg >= NB)
                    def _():
                        pltpu.make_async_copy(rows[b2], acc.at[dst_v.at[j]],
                                              ssems[b2]).wait()

                    pltpu.async_copy(t_hbm.at[src_v.at[jg]], rows[b2],
                                     gsems[b2])
            return carry

        lax.fori_loop(0, ch // NB, outer, 0)
        for b in range(NB):
            pltpu.make_async_copy(rows[b], acc.at[dst_v.at[b]],
                                  ssems[b]).wait()
        plsc.subcore_barrier()
        pltpu.sync_copy(acc.at[sl], out_hbm.at[cid, sl])

    return agg


def _proj_body(x_ref, w_ref, o_ref):
    o_ref[...] = jnp.dot(x_ref[...], w_ref[...],
                         preferred_element_type=jnp.float32)


def _proj(x, w, blk):
    n, f = x.shape
    d = w.shape[1]
    return pl.pallas_call(
        _proj_body,
        grid=(n // blk,),
        in_specs=[
            pl.BlockSpec((blk, f), lambda i: (i, 0)),
            pl.BlockSpec((f, d), lambda i: (0, 0)),
        ],
        out_specs=pl.BlockSpec((blk, d), lambda i: (i, 0)),
        out_shape=jax.ShapeDtypeStruct((n, d), jnp.float32),
    )(x, w)


def _mid_body(t_ref, p0_ref, p1_ref, ba_ref, wb_ref, bb_ref, g_ref, be_ref,
              wna_ref, o_ref):
    u = t_ref[...] + p0_ref[...] + p1_ref[...] + ba_ref[...]
    v = jnp.dot(jnp.maximum(u, 0.0), wb_ref[...],
                preferred_element_type=jnp.float32) + bb_ref[...]
    z = jnp.maximum(v, 0.0) * (g_ref[...] * BN_SCALE) + be_ref[...]
    o_ref[...] = jnp.dot(z, wna_ref[...], preferred_element_type=jnp.float32)


def _mid(t, p0, p1, ba, wb, bb, g, be, wna, blk):
    n, d = t.shape
    dn = wna.shape[1]
    vec = lambda: pl.BlockSpec((1, d), lambda i: (0, 0))
    mat = lambda a, b: pl.BlockSpec((a, b), lambda i: (0, 0))
    big = lambda: pl.BlockSpec((blk, d), lambda i: (i, 0))
    return pl.pallas_call(
        _mid_body,
        grid=(n // blk,),
        in_specs=[big(), big(), big(), vec(), mat(d, d), vec(), vec(), vec(),
                  mat(d, dn)],
        out_specs=pl.BlockSpec((blk, dn), lambda i: (i, 0)),
        out_shape=jax.ShapeDtypeStruct((n, dn), jnp.float32),
    )(t, p0, p1, ba.reshape(1, d), wb, bb.reshape(1, d), g.reshape(1, d),
      be.reshape(1, d), wna)


def _head_body(t_ref, p0_ref, p1_ref, ba_ref, wb_ref, bb_ref, g_ref, be_ref,
               wfc_ref, bfc_ref, o_ref):
    u = t_ref[...] + p0_ref[...] + p1_ref[...] + ba_ref[...]
    v = jnp.dot(jnp.maximum(u, 0.0), wb_ref[...],
                preferred_element_type=jnp.float32) + bb_ref[...]
    z = jnp.maximum(v, 0.0) * (g_ref[...] * BN_SCALE) + be_ref[...]
    logits = jnp.dot(z, wfc_ref[...],
                     preferred_element_type=jnp.float32) + bfc_ref[...]
    m = jnp.max(logits, axis=-1, keepdims=True)
    s = logits - m
    o_ref[...] = s - jnp.log(jnp.sum(jnp.exp(s), axis=-1, keepdims=True))


def _head(t, p0, p1, ba, wb, bb, g, be, wfc, bfc, blk):
    n, d = t.shape
    c = wfc.shape[1]
    vec = lambda w: pl.BlockSpec((1, w), lambda i: (0, 0))
    mat = lambda a, b: pl.BlockSpec((a, b), lambda i: (0, 0))
    big = lambda: pl.BlockSpec((blk, d), lambda i: (i, 0))
    return pl.pallas_call(
        _head_body,
        grid=(n // blk,),
        in_specs=[big(), big(), big(), vec(d), mat(d, d), vec(d), vec(d),
                  vec(d), mat(d, c), vec(c)],
        out_specs=pl.BlockSpec((blk, c), lambda i: (i, 0)),
        out_shape=jax.ShapeDtypeStruct((n, c), jnp.float32),
    )(t, p0, p1, ba.reshape(1, d), wb, bb.reshape(1, d), g.reshape(1, d),
      be.reshape(1, d), wfc, bfc.reshape(1, c))


def kernel(x, edge_index, batch, W1a, b1a, W1b, b1b, g1, be1, W2a, b2a, W2b,
           b2b, g2, be2, W3a, b3a, W3b, b3b, g3, be3, Wfc, bfc):
    n, f_in = x.shape
    e = edge_index.shape[1]
    dim = W1a.shape[1]

    n_pad = ((n + 1 + NS * 8 - 1) // (NS * 8)) * (NS * 8)
    ch = (e + NW * CB - 1) // (NW * CB)
    ch = ((ch + NB - 1) // NB) * NB
    e_pad = NW * ch * CB

    x_pad = jnp.pad(x, ((0, n_pad - n), (0, 0)))
    src = jnp.pad(edge_index[0], (0, e_pad - e), constant_values=n)
    dst = jnp.pad(edge_index[1], (0, e_pad - e), constant_values=n)
    srcm = src.reshape(NW, ch, CB)
    dstm = dst.reshape(NW, ch, CB)
    zeros = jnp.zeros((n_pad, dim), jnp.float32)

    sc_agg = _make_sc_agg(n_pad, ch, dim)
    blk = n_pad // 8

    t1 = _proj(x_pad, W1a, blk)
    p = sc_agg(t1, srcm, dstm, zeros)
    t2 = _mid(t1, p[0], p[1], b1a, W1b, b1b, g1, be1, W2a, blk)
    p = sc_agg(t2, srcm, dstm, zeros)
    t3 = _mid(t2, p[0], p[1], b2a, W2b, b2b, g2, be2, W3a, blk)
    p = sc_agg(t3, srcm, dstm, zeros)
    out = _head(t3, p[0], p[1], b3a, W3b, b3b, g3, be3, Wfc, bfc, blk)
    return out[:n]


# speedup vs baseline: 1.7472x; 1.7472x over previous
import functools
import math

import jax
import jax.numpy as jnp
from jax import lax
from jax.experimental import pallas as pl
from jax.experimental.pallas import tpu as pltpu
from jax.experimental.pallas import tpu_sc as plsc

NC = 2
NS = 16
NW = NC * NS
CB = 128

BN_SCALE = 1.0 / math.sqrt(1.0 + 1e-5)


NB = 12
NG = 9


def _make_sc_agg(n_pad, ch, dim):
    rpt = n_pad // NS
    mesh = plsc.VectorSubcoreMesh(core_axis_name="c", subcore_axis_name="s")

    @functools.partial(
        pl.kernel,
        out_type=jax.ShapeDtypeStruct((NC, n_pad, dim), jnp.float32),
        mesh=mesh,
        scratch_types=[
            pltpu.VMEM((ch, CB), jnp.int32),
            pltpu.VMEM((ch, CB), jnp.int32),
            [pltpu.VMEM((CB, dim), jnp.float32) for _ in range(NB)],
            [pltpu.SemaphoreType.DMA for _ in range(NB)],
            [pltpu.SemaphoreType.DMA for _ in range(NB)],
            pltpu.VMEM_SHARED((n_pad, dim), jnp.float32),
            pltpu.VMEM_SHARED((n_pad, dim), jnp.float32),
        ],
        compiler_params=pltpu.CompilerParams(use_tc_tiling_on_sc=False),
    )
    def agg(t_hbm, srcm, dstm, zeros_hbm, out_hbm, src_v, dst_v, rows, gsems,
            ssems, tbl, acc):
        cid = lax.axis_index("c")
        sid = lax.axis_index("s")
        wid = sid * NC + cid
        pltpu.sync_copy(srcm.at[wid], src_v)
        pltpu.sync_copy(dstm.at[wid], dst_v)
        sl = pl.ds(sid * rpt, rpt)
        pltpu.sync_copy(t_hbm.at[sl], tbl.at[sl])
        pltpu.sync_copy(zeros_hbm.at[sl], acc.at[sl])
        plsc.subcore_barrier()

        for b in range(NG):
            pltpu.async_copy(tbl.at[src_v.at[b]], rows[b], gsems[b])

        def outer(g, carry):
            for b in range(NB):
                j = g * NB + b
                pltpu.make_async_copy(tbl.at[src_v.at[b]], rows[b],
                                      gsems[b]).wait()
                pltpu.async_copy(rows[b], acc.at[dst_v.at[j]], ssems[b],
                                 add=True)
                b2 = (b + NG) % NB
                jg = j + NG

                @pl.when(jg < ch)
                def _():
                    @pl.when(jg >= NB)
                    def _():
                        pltpu.make_async_copy(rows[b2], acc.at[dst_v.at[j]],
                                              ssems[b2]).wait()

                    pltpu.async_copy(tbl.at[src_v.at[jg]], rows[b2],
                                     gsems[b2])
            return carry

        lax.fori_loop(0, ch // NB, outer, 0)
        for b in range(NB):
            pltpu.make_async_copy(rows[b], acc.at[dst_v.at[b]],
                                  ssems[b]).wait()
        plsc.subcore_barrier()
        pltpu.sync_copy(acc.at[sl], out_hbm.at[cid, sl])

    return agg


def _proj_body(x_ref, w_ref, o_ref):
    o_ref[...] = jnp.dot(x_ref[...], w_ref[...],
                         preferred_element_type=jnp.float32)


def _proj(x, w, blk):
    n, f = x.shape
    d = w.shape[1]
    return pl.pallas_call(
        _proj_body,
        grid=(n // blk,),
        in_specs=[
            pl.BlockSpec((blk, f), lambda i: (i, 0)),
            pl.BlockSpec((f, d), lambda i: (0, 0)),
        ],
        out_specs=pl.BlockSpec((blk, d), lambda i: (i, 0)),
        out_shape=jax.ShapeDtypeStruct((n, d), jnp.float32),
    )(x, w)


def _mid_body(t_ref, p0_ref, p1_ref, ba_ref, wb_ref, bb_ref, g_ref, be_ref,
              wna_ref, o_ref):
    u = t_ref[...] + p0_ref[...] + p1_ref[...] + ba_ref[...]
    v = jnp.dot(jnp.maximum(u, 0.0), wb_ref[...],
                preferred_element_type=jnp.float32) + bb_ref[...]
    z = jnp.maximum(v, 0.0) * (g_ref[...] * BN_SCALE) + be_ref[...]
    o_ref[...] = jnp.dot(z, wna_ref[...], preferred_element_type=jnp.float32)


def _mid(t, p0, p1, ba, wb, bb, g, be, wna, blk):
    n, d = t.shape
    dn = wna.shape[1]
    vec = lambda: pl.BlockSpec((1, d), lambda i: (0, 0))
    mat = lambda a, b: pl.BlockSpec((a, b), lambda i: (0, 0))
    big = lambda: pl.BlockSpec((blk, d), lambda i: (i, 0))
    return pl.pallas_call(
        _mid_body,
        grid=(n // blk,),
        in_specs=[big(), big(), big(), vec(), mat(d, d), vec(), vec(), vec(),
                  mat(d, dn)],
        out_specs=pl.BlockSpec((blk, dn), lambda i: (i, 0)),
        out_shape=jax.ShapeDtypeStruct((n, dn), jnp.float32),
    )(t, p0, p1, ba.reshape(1, d), wb, bb.reshape(1, d), g.reshape(1, d),
      be.reshape(1, d), wna)


def _head_body(t_ref, p0_ref, p1_ref, ba_ref, wb_ref, bb_ref, g_ref, be_ref,
               wfc_ref, bfc_ref, o_ref):
    u = t_ref[...] + p0_ref[...] + p1_ref[...] + ba_ref[...]
    v = jnp.dot(jnp.maximum(u, 0.0), wb_ref[...],
                preferred_element_type=jnp.float32) + bb_ref[...]
    z = jnp.maximum(v, 0.0) * (g_ref[...] * BN_SCALE) + be_ref[...]
    logits = jnp.dot(z, wfc_ref[...],
                     preferred_element_type=jnp.float32) + bfc_ref[...]
    m = jnp.max(logits, axis=-1, keepdims=True)
    s = logits - m
    o_ref[...] = s - jnp.log(jnp.sum(jnp.exp(s), axis=-1, keepdims=True))


def _head(t, p0, p1, ba, wb, bb, g, be, wfc, bfc, blk):
    n, d = t.shape
    c = wfc.shape[1]
    vec = lambda w: pl.BlockSpec((1, w), lambda i: (0, 0))
    mat = lambda a, b: pl.BlockSpec((a, b), lambda i: (0, 0))
    big = lambda: pl.BlockSpec((blk, d), lambda i: (i, 0))
    return pl.pallas_call(
        _head_body,
        grid=(n // blk,),
        in_specs=[big(), big(), big(), vec(d), mat(d, d), vec(d), vec(d),
                  vec(d), mat(d, c), vec(c)],
        out_specs=pl.BlockSpec((blk, c), lambda i: (i, 0)),
        out_shape=jax.ShapeDtypeStruct((n, c), jnp.float32),
    )(t, p0, p1, ba.reshape(1, d), wb, bb.reshape(1, d), g.reshape(1, d),
      be.reshape(1, d), wfc, bfc.reshape(1, c))


def kernel(x, edge_index, batch, W1a, b1a, W1b, b1b, g1, be1, W2a, b2a, W2b,
           b2b, g2, be2, W3a, b3a, W3b, b3b, g3, be3, Wfc, bfc):
    n, f_in = x.shape
    e = edge_index.shape[1]
    dim = W1a.shape[1]

    n_pad = ((n + 1 + NS * 8 - 1) // (NS * 8)) * (NS * 8)
    ch = (e + NW * CB - 1) // (NW * CB)
    ch = ((ch + NB - 1) // NB) * NB
    e_pad = NW * ch * CB

    x_pad = jnp.pad(x, ((0, n_pad - n), (0, 0)))
    src = jnp.pad(edge_index[0], (0, e_pad - e), constant_values=n)
    dst = jnp.pad(edge_index[1], (0, e_pad - e), constant_values=n)
    srcm = src.reshape(NW, ch, CB)
    dstm = dst.reshape(NW, ch, CB)
    zeros = jnp.zeros((n_pad, dim), jnp.float32)

    sc_agg = _make_sc_agg(n_pad, ch, dim)
    blk = n_pad // 8

    t1 = _proj(x_pad, W1a, blk)
    p = sc_agg(t1, srcm, dstm, zeros)
    t2 = _mid(t1, p[0], p[1], b1a, W1b, b1b, g1, be1, W2a, blk)
    p = sc_agg(t2, srcm, dstm, zeros)
    t3 = _mid(t2, p[0], p[1], b2a, W2b, b2b, g2, be2, W3a, blk)
    p = sc_agg(t3, srcm, dstm, zeros)
    out = _head(t3, p[0], p[1], b3a, W3b, b3b, g3, be3, Wfc, bfc, blk)
    return out[:n]


# speedup vs baseline: 1.8769x; 1.0742x over previous
import functools
import math

import jax
import jax.numpy as jnp
from jax import lax
from jax.experimental import pallas as pl
from jax.experimental.pallas import tpu as pltpu
from jax.experimental.pallas import tpu_sc as plsc

NC = 2
NS = 16
NW = NC * NS
CB = 128

BN_SCALE = 1.0 / math.sqrt(1.0 + 1e-5)


NB = 10
NG = 8


def _make_sc_agg(n_pad, ch, dim):
    rpt = n_pad // NS
    mesh = plsc.VectorSubcoreMesh(core_axis_name="c", subcore_axis_name="s")

    @functools.partial(
        pl.kernel,
        out_type=jax.ShapeDtypeStruct((NC, n_pad, dim), jnp.float32),
        mesh=mesh,
        scratch_types=[
            pltpu.VMEM((ch, CB), jnp.int32),
            pltpu.VMEM((ch, CB), jnp.int32),
            [pltpu.VMEM((CB, dim), jnp.float32) for _ in range(NB)],
            [pltpu.SemaphoreType.DMA for _ in range(NB)],
            [pltpu.SemaphoreType.DMA for _ in range(NB)],
            pltpu.VMEM_SHARED((n_pad, dim), jnp.float32),
            pltpu.VMEM_SHARED((n_pad, dim), jnp.float32),
        ],
        compiler_params=pltpu.CompilerParams(use_tc_tiling_on_sc=False),
    )
    def agg(t_hbm, srcm, dstm, zeros_hbm, out_hbm, src_v, dst_v, rows, gsems,
            ssems, tbl, acc):
        cid = lax.axis_index("c")
        sid = lax.axis_index("s")
        wid = sid * NC + cid
        pltpu.sync_copy(srcm.at[wid], src_v)
        pltpu.sync_copy(dstm.at[wid], dst_v)
        sl = pl.ds(sid * rpt, rpt)
        pltpu.sync_copy(t_hbm.at[sl], tbl.at[sl])
        pltpu.sync_copy(zeros_hbm.at[sl], acc.at[sl])
        plsc.subcore_barrier()

        for b in range(NG):
            pltpu.async_copy(tbl.at[src_v.at[b]], rows[b], gsems[b])

        def outer(g, carry):
            for b in range(NB):
                j = g * NB + b
                pltpu.make_async_copy(tbl.at[src_v.at[b]], rows[b],
                                      gsems[b]).wait()
                pltpu.async_copy(rows[b], acc.at[dst_v.at[j]], ssems[b],
                                 add=True)
                b2 = (b + NG) % NB
                jg = j + NG

                @pl.when(jg < ch)
                def _():
                    @pl.when(jg >= NB)
                    def _():
                        pltpu.make_async_copy(rows[b2], acc.at[dst_v.at[j]],
                                              ssems[b2]).wait()

                    pltpu.async_copy(tbl.at[src_v.at[jg]], rows[b2],
                                     gsems[b2])
            return carry

        lax.fori_loop(0, ch // NB, outer, 0)
        for b in range(NB):
            pltpu.make_async_copy(rows[b], acc.at[dst_v.at[b]],
                                  ssems[b]).wait()
        plsc.subcore_barrier()
        pltpu.sync_copy(acc.at[sl], out_hbm.at[cid, sl])

    return agg


def _proj_body(x_ref, w_ref, o_ref):
    o_ref[...] = jnp.dot(x_ref[...], w_ref[...],
                         preferred_element_type=jnp.float32)


def _proj(x, w, blk):
    n, f = x.shape
    d = w.shape[1]
    return pl.pallas_call(
        _proj_body,
        grid=(n // blk,),
        in_specs=[
            pl.BlockSpec((blk, f), lambda i: (i, 0)),
            pl.BlockSpec((f, d), lambda i: (0, 0)),
        ],
        out_specs=pl.BlockSpec((blk, d), lambda i: (i, 0)),
        out_shape=jax.ShapeDtypeStruct((n, d), jnp.float32),
    )(x, w)


def _mid_body(t_ref, p0_ref, p1_ref, ba_ref, wb_ref, bb_ref, g_ref, be_ref,
              wna_ref, o_ref):
    u = t_ref[...] + p0_ref[...] + p1_ref[...] + ba_ref[...]
    v = jnp.dot(jnp.maximum(u, 0.0), wb_ref[...],
                preferred_element_type=jnp.float32) + bb_ref[...]
    z = jnp.maximum(v, 0.0) * (g_ref[...] * BN_SCALE) + be_ref[...]
    o_ref[...] = jnp.dot(z, wna_ref[...], preferred_element_type=jnp.float32)


def _mid(t, p0, p1, ba, wb, bb, g, be, wna, blk):
    n, d = t.shape
    dn = wna.shape[1]
    vec = lambda: pl.BlockSpec((1, d), lambda i: (0, 0))
    mat = lambda a, b: pl.BlockSpec((a, b), lambda i: (0, 0))
    big = lambda: pl.BlockSpec((blk, d), lambda i: (i, 0))
    return pl.pallas_call(
        _mid_body,
        grid=(n // blk,),
        in_specs=[big(), big(), big(), vec(), mat(d, d), vec(), vec(), vec(),
                  mat(d, dn)],
        out_specs=pl.BlockSpec((blk, dn), lambda i: (i, 0)),
        out_shape=jax.ShapeDtypeStruct((n, dn), jnp.float32),
    )(t, p0, p1, ba.reshape(1, d), wb, bb.reshape(1, d), g.reshape(1, d),
      be.reshape(1, d), wna)


def _head_body(t_ref, p0_ref, p1_ref, ba_ref, wb_ref, bb_ref, g_ref, be_ref,
               wfc_ref, bfc_ref, o_ref):
    u = t_ref[...] + p0_ref[...] + p1_ref[...] + ba_ref[...]
    v = jnp.dot(jnp.maximum(u, 0.0), wb_ref[...],
                preferred_element_type=jnp.float32) + bb_ref[...]
    z = jnp.maximum(v, 0.0) * (g_ref[...] * BN_SCALE) + be_ref[...]
    logits = jnp.dot(z, wfc_ref[...],
                     preferred_element_type=jnp.float32) + bfc_ref[...]
    m = jnp.max(logits, axis=-1, keepdims=True)
    s = logits - m
    o_ref[...] = s - jnp.log(jnp.sum(jnp.exp(s), axis=-1, keepdims=True))


def _head(t, p0, p1, ba, wb, bb, g, be, wfc, bfc, blk):
    n, d = t.shape
    c = wfc.shape[1]
    vec = lambda w: pl.BlockSpec((1, w), lambda i: (0, 0))
    mat = lambda a, b: pl.BlockSpec((a, b), lambda i: (0, 0))
    big = lambda: pl.BlockSpec((blk, d), lambda i: (i, 0))
    return pl.pallas_call(
        _head_body,
        grid=(n // blk,),
        in_specs=[big(), big(), big(), vec(d), mat(d, d), vec(d), vec(d),
                  vec(d), mat(d, c), vec(c)],
        out_specs=pl.BlockSpec((blk, c), lambda i: (i, 0)),
        out_shape=jax.ShapeDtypeStruct((n, c), jnp.float32),
    )(t, p0, p1, ba.reshape(1, d), wb, bb.reshape(1, d), g.reshape(1, d),
      be.reshape(1, d), wfc, bfc.reshape(1, c))


def kernel(x, edge_index, batch, W1a, b1a, W1b, b1b, g1, be1, W2a, b2a, W2b,
           b2b, g2, be2, W3a, b3a, W3b, b3b, g3, be3, Wfc, bfc):
    n, f_in = x.shape
    e = edge_index.shape[1]
    dim = W1a.shape[1]

    n_pad = ((n + 1 + NS * 8 - 1) // (NS * 8)) * (NS * 8)
    ch = (e + NW * CB - 1) // (NW * CB)
    ch = ((ch + NB - 1) // NB) * NB
    e_pad = NW * ch * CB

    x_pad = jnp.pad(x, ((0, n_pad - n), (0, 0)))
    src = jnp.pad(edge_index[0], (0, e_pad - e), constant_values=n)
    dst = jnp.pad(edge_index[1], (0, e_pad - e), constant_values=n)
    srcm = src.reshape(NW, ch, CB)
    dstm = dst.reshape(NW, ch, CB)
    zeros = jnp.zeros((n_pad, dim), jnp.float32)

    sc_agg = _make_sc_agg(n_pad, ch, dim)
    blk = n_pad // 8

    t1 = _proj(x_pad, W1a, blk)
    p = sc_agg(t1, srcm, dstm, zeros)
    t2 = _mid(t1, p[0], p[1], b1a, W1b, b1b, g1, be1, W2a, blk)
    p = sc_agg(t2, srcm, dstm, zeros)
    t3 = _mid(t2, p[0], p[1], b2a, W2b, b2b, g2, be2, W3a, blk)
    p = sc_agg(t3, srcm, dstm, zeros)
    out = _head(t3, p[0], p[1], b3a, W3b, b3b, g3, be3, Wfc, bfc, blk)
    return out[:n]


# speedup vs baseline: 1.9490x; 1.0384x over previous
import functools
import math

import jax
import jax.numpy as jnp
from jax import lax
from jax.experimental import pallas as pl
from jax.experimental.pallas import tpu as pltpu
from jax.experimental.pallas import tpu_sc as plsc

NC = 2
NS = 16
NW = NC * NS
CB = 128

BN_SCALE = 1.0 / math.sqrt(1.0 + 1e-5)


NB = 10
NG = 8


def _make_sc_agg(n_pad, ch, dim):
    rpt = n_pad // NS
    mesh = plsc.VectorSubcoreMesh(core_axis_name="c", subcore_axis_name="s")

    @functools.partial(
        pl.kernel,
        out_type=jax.ShapeDtypeStruct((NC, n_pad, dim), jnp.float32),
        mesh=mesh,
        scratch_types=[
            pltpu.VMEM((ch, CB), jnp.int32),
            pltpu.VMEM((ch, CB), jnp.int32),
            [pltpu.VMEM((CB, dim), jnp.float32) for _ in range(NB)],
            [pltpu.SemaphoreType.DMA for _ in range(NB)],
            [pltpu.SemaphoreType.DMA for _ in range(NB)],
            pltpu.VMEM_SHARED((n_pad, dim), jnp.float32),
            pltpu.VMEM_SHARED((n_pad, dim), jnp.float32),
        ],
        compiler_params=pltpu.CompilerParams(use_tc_tiling_on_sc=False),
    )
    def agg(t_hbm, srcm, dstm, zeros_hbm, out_hbm, src_v, dst_v, rows, gsems,
            ssems, tbl, acc):
        cid = lax.axis_index("c")
        sid = lax.axis_index("s")
        wid = sid * NC + cid
        sl = pl.ds(sid * rpt, rpt)
        c0 = pltpu.async_copy(srcm.at[wid], src_v, gsems[0])
        c1 = pltpu.async_copy(dstm.at[wid], dst_v, gsems[1])
        c2 = pltpu.async_copy(t_hbm.at[sl], tbl.at[sl], gsems[2])
        c3 = pltpu.async_copy(zeros_hbm.at[sl], acc.at[sl], gsems[3])
        c0.wait(); c1.wait(); c2.wait(); c3.wait()
        plsc.subcore_barrier()

        for b in range(NG):
            pltpu.async_copy(tbl.at[src_v.at[b]], rows[b], gsems[b])

        def outer(g, carry):
            for b in range(NB):
                j = g * NB + b
                pltpu.make_async_copy(tbl.at[src_v.at[b]], rows[b],
                                      gsems[b]).wait()
                pltpu.async_copy(rows[b], acc.at[dst_v.at[j]], ssems[b],
                                 add=True)
                b2 = (b + NG) % NB
                jg = j + NG

                @pl.when(jg < ch)
                def _():
                    @pl.when(jg >= NB)
                    def _():
                        pltpu.make_async_copy(rows[b2], acc.at[dst_v.at[j]],
                                              ssems[b2]).wait()

                    pltpu.async_copy(tbl.at[src_v.at[jg]], rows[b2],
                                     gsems[b2])
            return carry

        lax.fori_loop(0, ch // NB, outer, 0)
        for b in range(NB):
            pltpu.make_async_copy(rows[b], acc.at[dst_v.at[b]],
                                  ssems[b]).wait()
        plsc.subcore_barrier()
        pltpu.sync_copy(acc.at[sl], out_hbm.at[cid, sl])

    return agg


def _proj_body(x_ref, w_ref, o_ref):
    o_ref[...] = jnp.dot(x_ref[...], w_ref[...],
                         preferred_element_type=jnp.float32)


def _proj(x, w, n_out, nblk):
    n, f = x.shape
    d = w.shape[1]
    blk = n // nblk
    return pl.pallas_call(
        _proj_body,
        grid=(nblk,),
        in_specs=[
            pl.BlockSpec((blk, f), lambda i: (i, 0)),
            pl.BlockSpec((f, d), lambda i: (0, 0)),
        ],
        out_specs=pl.BlockSpec((blk, d), lambda i: (i, 0)),
        out_shape=jax.ShapeDtypeStruct((n_out, d), jnp.float32),
    )(x, w)


def _mid_body(t_ref, p0_ref, p1_ref, ba_ref, wb_ref, bb_ref, g_ref, be_ref,
              wna_ref, o_ref):
    u = t_ref[...] + p0_ref[...] + p1_ref[...] + ba_ref[...]
    v = jnp.dot(jnp.maximum(u, 0.0), wb_ref[...],
                preferred_element_type=jnp.float32) + bb_ref[...]
    z = jnp.maximum(v, 0.0) * (g_ref[...] * BN_SCALE) + be_ref[...]
    o_ref[...] = jnp.dot(z, wna_ref[...], preferred_element_type=jnp.float32)


def _mid(t, p0, p1, ba, wb, bb, g, be, wna, blk):
    n, d = t.shape
    dn = wna.shape[1]
    vec = lambda: pl.BlockSpec((1, d), lambda i: (0, 0))
    mat = lambda a, b: pl.BlockSpec((a, b), lambda i: (0, 0))
    big = lambda: pl.BlockSpec((blk, d), lambda i: (i, 0))
    return pl.pallas_call(
        _mid_body,
        grid=(n // blk,),
        in_specs=[big(), big(), big(), vec(), mat(d, d), vec(), vec(), vec(),
                  mat(d, dn)],
        out_specs=pl.BlockSpec((blk, dn), lambda i: (i, 0)),
        out_shape=jax.ShapeDtypeStruct((n, dn), jnp.float32),
    )(t, p0, p1, ba.reshape(1, d), wb, bb.reshape(1, d), g.reshape(1, d),
      be.reshape(1, d), wna)


def _head_body(t_ref, p0_ref, p1_ref, ba_ref, wb_ref, bb_ref, g_ref, be_ref,
               wfc_ref, bfc_ref, o_ref):
    u = t_ref[...] + p0_ref[...] + p1_ref[...] + ba_ref[...]
    v = jnp.dot(jnp.maximum(u, 0.0), wb_ref[...],
                preferred_element_type=jnp.float32) + bb_ref[...]
    z = jnp.maximum(v, 0.0) * (g_ref[...] * BN_SCALE) + be_ref[...]
    logits = jnp.dot(z, wfc_ref[...],
                     preferred_element_type=jnp.float32) + bfc_ref[...]
    m = jnp.max(logits, axis=-1, keepdims=True)
    s = logits - m
    o_ref[...] = s - jnp.log(jnp.sum(jnp.exp(s), axis=-1, keepdims=True))


def _head(t, p0, p1, ba, wb, bb, g, be, wfc, bfc, blk):
    n, d = t.shape
    c = wfc.shape[1]
    vec = lambda w: pl.BlockSpec((1, w), lambda i: (0, 0))
    mat = lambda a, b: pl.BlockSpec((a, b), lambda i: (0, 0))
    big = lambda: pl.BlockSpec((blk, d), lambda i: (i, 0))
    return pl.pallas_call(
        _head_body,
        grid=(n // blk,),
        in_specs=[big(), big(), big(), vec(d), mat(d, d), vec(d), vec(d),
                  vec(d), mat(d, c), vec(c)],
        out_specs=pl.BlockSpec((blk, c), lambda i: (i, 0)),
        out_shape=jax.ShapeDtypeStruct((n, c), jnp.float32),
    )(t, p0, p1, ba.reshape(1, d), wb, bb.reshape(1, d), g.reshape(1, d),
      be.reshape(1, d), wfc, bfc.reshape(1, c))


def kernel(x, edge_index, batch, W1a, b1a, W1b, b1b, g1, be1, W2a, b2a, W2b,
           b2b, g2, be2, W3a, b3a, W3b, b3b, g3, be3, Wfc, bfc):
    n, f_in = x.shape
    e = edge_index.shape[1]
    dim = W1a.shape[1]

    n_pad = ((n + 1 + NS * 8 - 1) // (NS * 8)) * (NS * 8)
    ch = (e + NW * CB - 1) // (NW * CB)
    ch = ((ch + NB - 1) // NB) * NB
    e_pad = NW * ch * CB

    src = jnp.pad(edge_index[0], (0, e_pad - e), constant_values=n)
    dst = jnp.pad(edge_index[1], (0, e_pad - e), constant_values=n)
    srcm = src.reshape(NW, ch, CB)
    dstm = dst.reshape(NW, ch, CB)
    zeros = jnp.zeros((n_pad, dim), jnp.float32)

    sc_agg = _make_sc_agg(n_pad, ch, dim)
    blk = n_pad // 8

    t1 = _proj(x, W1a, n_pad, 5)
    p = sc_agg(t1, srcm, dstm, zeros)
    t2 = _mid(t1, p[0], p[1], b1a, W1b, b1b, g1, be1, W2a, blk)
    p = sc_agg(t2, srcm, dstm, zeros)
    t3 = _mid(t2, p[0], p[1], b2a, W2b, b2b, g2, be2, W3a, blk)
    p = sc_agg(t3, srcm, dstm, zeros)
    out = _head(t3, p[0], p[1], b3a, W3b, b3b, g3, be3, Wfc, bfc, blk)
    return out[:n]


# speedup vs baseline: 1.9662x; 1.0089x over previous
import functools
import math

import jax
import jax.numpy as jnp
from jax import lax
from jax.experimental import pallas as pl
from jax.experimental.pallas import tpu as pltpu
from jax.experimental.pallas import tpu_sc as plsc

NC = 2
NS = 16
NW = NC * NS
CB = 128

BN_SCALE = 1.0 / math.sqrt(1.0 + 1e-5)


NB = 10
NG = 8


def _make_sc_agg(n_pad, ch, dim):
    rpt = n_pad // NS
    mesh = plsc.VectorSubcoreMesh(core_axis_name="c", subcore_axis_name="s")

    @functools.partial(
        pl.kernel,
        out_type=jax.ShapeDtypeStruct((NC, n_pad, dim), jnp.float32),
        mesh=mesh,
        scratch_types=[
            pltpu.VMEM((ch, CB), jnp.int32),
            pltpu.VMEM((ch, CB), jnp.int32),
            [pltpu.VMEM((CB, dim), jnp.float32) for _ in range(NB)],
            [pltpu.SemaphoreType.DMA for _ in range(NB)],
            [pltpu.SemaphoreType.DMA for _ in range(NB)],
            pltpu.VMEM_SHARED((n_pad, dim), jnp.float32),
            pltpu.VMEM_SHARED((n_pad, dim), jnp.float32),
        ],
        compiler_params=pltpu.CompilerParams(use_tc_tiling_on_sc=False),
    )
    def agg(t_hbm, srcm, dstm, zeros_hbm, out_hbm, src_v, dst_v, rows, gsems,
            ssems, tbl, acc):
        cid = lax.axis_index("c")
        sid = lax.axis_index("s")
        wid = sid * NC + cid
        sl = pl.ds(sid * rpt, rpt)
        c0 = pltpu.async_copy(srcm.at[wid], src_v, gsems[0])
        c1 = pltpu.async_copy(dstm.at[wid], dst_v, gsems[1])
        c2 = pltpu.async_copy(t_hbm.at[sl], tbl.at[sl], gsems[2])

        @pl.when(cid == 0)
        def _():
            pltpu.async_copy(t_hbm.at[sl], acc.at[sl], ssems[0])

        @pl.when(cid == 1)
        def _():
            pltpu.async_copy(zeros_hbm.at[sl], acc.at[sl], ssems[0])

        c0.wait(); c2.wait()
        plsc.subcore_barrier()

        for b in range(NG):
            pltpu.async_copy(tbl.at[src_v.at[b]], rows[b], gsems[b])

        c1.wait()
        pltpu.make_async_copy(zeros_hbm.at[sl], acc.at[sl], ssems[0]).wait()
        plsc.subcore_barrier()

        def outer(g, carry):
            for b in range(NB):
                j = g * NB + b
                pltpu.make_async_copy(tbl.at[src_v.at[b]], rows[b],
                                      gsems[b]).wait()
                pltpu.async_copy(rows[b], acc.at[dst_v.at[j]], ssems[b],
                                 add=True)
                b2 = (b + NG) % NB
                jg = j + NG

                @pl.when(jg < ch)
                def _():
                    @pl.when(jg >= NB)
                    def _():
                        pltpu.make_async_copy(rows[b2], acc.at[dst_v.at[j]],
                                              ssems[b2]).wait()

                    pltpu.async_copy(tbl.at[src_v.at[jg]], rows[b2],
                                     gsems[b2])
            return carry

        lax.fori_loop(0, ch // NB, outer, 0)
        for b in range(NB):
            pltpu.make_async_copy(rows[b], acc.at[dst_v.at[b]],
                                  ssems[b]).wait()
        plsc.subcore_barrier()
        pltpu.sync_copy(acc.at[sl], out_hbm.at[cid, sl])

    return agg


def _proj_body(x_ref, w_ref, o_ref):
    o_ref[...] = jnp.dot(x_ref[...], w_ref[...],
                         preferred_element_type=jnp.float32)


def _proj(x, w, n_out, nblk):
    n, f = x.shape
    d = w.shape[1]
    blk = n // nblk
    return pl.pallas_call(
        _proj_body,
        grid=(nblk,),
        in_specs=[
            pl.BlockSpec((blk, f), lambda i: (i, 0)),
            pl.BlockSpec((f, d), lambda i: (0, 0)),
        ],
        out_specs=pl.BlockSpec((blk, d), lambda i: (i, 0)),
        out_shape=jax.ShapeDtypeStruct((n_out, d), jnp.float32),
    )(x, w)


def _mid_body(p0_ref, p1_ref, ba_ref, wb_ref, bb_ref, g_ref, be_ref,
              wna_ref, o_ref):
    u = p0_ref[...] + p1_ref[...] + ba_ref[...]
    v = jnp.dot(jnp.maximum(u, 0.0), wb_ref[...],
                preferred_element_type=jnp.float32) + bb_ref[...]
    z = jnp.maximum(v, 0.0) * (g_ref[...] * BN_SCALE) + be_ref[...]
    o_ref[...] = jnp.dot(z, wna_ref[...], preferred_element_type=jnp.float32)


def _mid(p0, p1, ba, wb, bb, g, be, wna, blk):
    n, d = p0.shape
    dn = wna.shape[1]
    vec = lambda: pl.BlockSpec((1, d), lambda i: (0, 0))
    mat = lambda a, b: pl.BlockSpec((a, b), lambda i: (0, 0))
    big = lambda: pl.BlockSpec((blk, d), lambda i: (i, 0))
    return pl.pallas_call(
        _mid_body,
        grid=(n // blk,),
        in_specs=[big(), big(), vec(), mat(d, d), vec(), vec(), vec(),
                  mat(d, dn)],
        out_specs=pl.BlockSpec((blk, dn), lambda i: (i, 0)),
        out_shape=jax.ShapeDtypeStruct((n, dn), jnp.float32),
    )(p0, p1, ba.reshape(1, d), wb, bb.reshape(1, d), g.reshape(1, d),
      be.reshape(1, d), wna)


def _head_body(p0_ref, p1_ref, ba_ref, wb_ref, bb_ref, g_ref, be_ref,
               wfc_ref, bfc_ref, o_ref):
    u = p0_ref[...] + p1_ref[...] + ba_ref[...]
    v = jnp.dot(jnp.maximum(u, 0.0), wb_ref[...],
                preferred_element_type=jnp.float32) + bb_ref[...]
    z = jnp.maximum(v, 0.0) * (g_ref[...] * BN_SCALE) + be_ref[...]
    logits = jnp.dot(z, wfc_ref[...],
                     preferred_element_type=jnp.float32) + bfc_ref[...]
    m = jnp.max(logits, axis=-1, keepdims=True)
    s = logits - m
    o_ref[...] = s - jnp.log(jnp.sum(jnp.exp(s), axis=-1, keepdims=True))


def _head(p0, p1, ba, wb, bb, g, be, wfc, bfc, blk):
    n, d = p0.shape
    c = wfc.shape[1]
    vec = lambda w: pl.BlockSpec((1, w), lambda i: (0, 0))
    mat = lambda a, b: pl.BlockSpec((a, b), lambda i: (0, 0))
    big = lambda: pl.BlockSpec((blk, d), lambda i: (i, 0))
    return pl.pallas_call(
        _head_body,
        grid=(n // blk,),
        in_specs=[big(), big(), vec(d), mat(d, d), vec(d), vec(d),
                  vec(d), mat(d, c), vec(c)],
        out_specs=pl.BlockSpec((blk, c), lambda i: (i, 0)),
        out_shape=jax.ShapeDtypeStruct((n, c), jnp.float32),
    )(p0, p1, ba.reshape(1, d), wb, bb.reshape(1, d), g.reshape(1, d),
      be.reshape(1, d), wfc, bfc.reshape(1, c))


def kernel(x, edge_index, batch, W1a, b1a, W1b, b1b, g1, be1, W2a, b2a, W2b,
           b2b, g2, be2, W3a, b3a, W3b, b3b, g3, be3, Wfc, bfc):
    n, f_in = x.shape
    e = edge_index.shape[1]
    dim = W1a.shape[1]

    n_pad = ((n + 1 + NS * 8 - 1) // (NS * 8)) * (NS * 8)
    ch = (e + NW * CB - 1) // (NW * CB)
    ch = ((ch + NB - 1) // NB) * NB
    e_pad = NW * ch * CB

    src = jnp.pad(edge_index[0], (0, e_pad - e), constant_values=n)
    dst = jnp.pad(edge_index[1], (0, e_pad - e), constant_values=n)
    srcm = src.reshape(NW, ch, CB)
    dstm = dst.reshape(NW, ch, CB)
    zeros = jnp.zeros((n_pad, dim), jnp.float32)

    sc_agg = _make_sc_agg(n_pad, ch, dim)
    blk = n_pad // 8

    t1 = _proj(x, W1a, n_pad, 5)
    p = sc_agg(t1, srcm, dstm, zeros)
    t2 = _mid(p[0], p[1], b1a, W1b, b1b, g1, be1, W2a, blk)
    p = sc_agg(t2, srcm, dstm, zeros)
    t3 = _mid(p[0], p[1], b2a, W2b, b2b, g2, be2, W3a, blk)
    p = sc_agg(t3, srcm, dstm, zeros)
    out = _head(p[0], p[1], b3a, W3b, b3b, g3, be3, Wfc, bfc, blk)
    return out[:n]


# speedup vs baseline: 1.9672x; 1.0005x over previous
import functools
import math

import jax
import jax.numpy as jnp
from jax import lax
from jax.experimental import pallas as pl
from jax.experimental.pallas import tpu as pltpu
from jax.experimental.pallas import tpu_sc as plsc

NC = 2
NS = 16
NW = NC * NS
CB = 128

BN_SCALE = 1.0 / math.sqrt(1.0 + 1e-5)


NB = 10
NG = 9


def _make_sc_agg(n_pad, ch, dim):
    rpt = n_pad // NS
    mesh = plsc.VectorSubcoreMesh(core_axis_name="c", subcore_axis_name="s")

    @functools.partial(
        pl.kernel,
        out_type=jax.ShapeDtypeStruct((NC, n_pad, dim), jnp.float32),
        mesh=mesh,
        scratch_types=[
            pltpu.VMEM((ch, CB), jnp.int32),
            pltpu.VMEM((ch, CB), jnp.int32),
            [pltpu.VMEM((CB, dim), jnp.float32) for _ in range(NB)],
            [pltpu.SemaphoreType.DMA for _ in range(NB)],
            [pltpu.SemaphoreType.DMA for _ in range(NB)],
            pltpu.VMEM_SHARED((n_pad, dim), jnp.float32),
            pltpu.VMEM_SHARED((n_pad, dim), jnp.float32),
        ],
        compiler_params=pltpu.CompilerParams(use_tc_tiling_on_sc=False),
    )
    def agg(t_hbm, srcm, dstm, zeros_hbm, out_hbm, src_v, dst_v, rows, gsems,
            ssems, tbl, acc):
        cid = lax.axis_index("c")
        sid = lax.axis_index("s")
        wid = sid * NC + cid
        sl = pl.ds(sid * rpt, rpt)
        c0 = pltpu.async_copy(srcm.at[wid], src_v, gsems[0])
        c1 = pltpu.async_copy(dstm.at[wid], dst_v, gsems[1])
        c2 = pltpu.async_copy(t_hbm.at[sl], tbl.at[sl], gsems[2])

        @pl.when(cid == 0)
        def _():
            pltpu.async_copy(t_hbm.at[sl], acc.at[sl], ssems[0])

        @pl.when(cid == 1)
        def _():
            pltpu.async_copy(zeros_hbm.at[sl], acc.at[sl], ssems[0])

        c0.wait(); c2.wait()
        plsc.subcore_barrier()

        for b in range(NG):
            pltpu.async_copy(tbl.at[src_v.at[b]], rows[b], gsems[b])

        c1.wait()
        pltpu.make_async_copy(zeros_hbm.at[sl], acc.at[sl], ssems[0]).wait()
        plsc.subcore_barrier()

        def outer(g, carry):
            for b in range(NB):
                j = g * NB + b
                pltpu.make_async_copy(tbl.at[src_v.at[b]], rows[b],
                                      gsems[b]).wait()
                pltpu.async_copy(rows[b], acc.at[dst_v.at[j]], ssems[b],
                                 add=True)
                b2 = (b + NG) % NB
                jg = j + NG

                @pl.when(jg < ch)
                def _():
                    @pl.when(jg >= NB)
                    def _():
                        pltpu.make_async_copy(rows[b2], acc.at[dst_v.at[j]],
                                              ssems[b2]).wait()

                    pltpu.async_copy(tbl.at[src_v.at[jg]], rows[b2],
                                     gsems[b2])
            return carry

        lax.fori_loop(0, ch // NB, outer, 0)
        for b in range(NB):
            pltpu.make_async_copy(rows[b], acc.at[dst_v.at[b]],
                                  ssems[b]).wait()
        plsc.subcore_barrier()
        pltpu.sync_copy(acc.at[sl], out_hbm.at[cid, sl])

    return agg


def _proj_body(x_ref, w_ref, o_ref):
    o_ref[...] = jnp.dot(x_ref[...], w_ref[...],
                         preferred_element_type=jnp.float32)


def _proj(x, w, n_out, nblk):
    n, f = x.shape
    d = w.shape[1]
    blk = n // nblk
    return pl.pallas_call(
        _proj_body,
        grid=(nblk,),
        in_specs=[
            pl.BlockSpec((blk, f), lambda i: (i, 0)),
            pl.BlockSpec((f, d), lambda i: (0, 0)),
        ],
        out_specs=pl.BlockSpec((blk, d), lambda i: (i, 0)),
        out_shape=jax.ShapeDtypeStruct((n_out, d), jnp.float32),
    )(x, w)


def _mid_body(p0_ref, p1_ref, ba_ref, wb_ref, bb_ref, g_ref, be_ref,
              wna_ref, o_ref):
    u = p0_ref[...] + p1_ref[...] + ba_ref[...]
    v = jnp.dot(jnp.maximum(u, 0.0), wb_ref[...],
                preferred_element_type=jnp.float32) + bb_ref[...]
    z = jnp.maximum(v, 0.0) * (g_ref[...] * BN_SCALE) + be_ref[...]
    o_ref[...] = jnp.dot(z, wna_ref[...], preferred_element_type=jnp.float32)


def _mid(p0, p1, ba, wb, bb, g, be, wna, blk):
    n, d = p0.shape
    dn = wna.shape[1]
    vec = lambda: pl.BlockSpec((1, d), lambda i: (0, 0))
    mat = lambda a, b: pl.BlockSpec((a, b), lambda i: (0, 0))
    big = lambda: pl.BlockSpec((blk, d), lambda i: (i, 0))
    return pl.pallas_call(
        _mid_body,
        grid=(n // blk,),
        in_specs=[big(), big(), vec(), mat(d, d), vec(), vec(), vec(),
                  mat(d, dn)],
        out_specs=pl.BlockSpec((blk, dn), lambda i: (i, 0)),
        out_shape=jax.ShapeDtypeStruct((n, dn), jnp.float32),
    )(p0, p1, ba.reshape(1, d), wb, bb.reshape(1, d), g.reshape(1, d),
      be.reshape(1, d), wna)


def _head_body(p0_ref, p1_ref, ba_ref, wb_ref, bb_ref, g_ref, be_ref,
               wfc_ref, bfc_ref, o_ref):
    u = p0_ref[...] + p1_ref[...] + ba_ref[...]
    v = jnp.dot(jnp.maximum(u, 0.0), wb_ref[...],
                preferred_element_type=jnp.float32) + bb_ref[...]
    z = jnp.maximum(v, 0.0) * (g_ref[...] * BN_SCALE) + be_ref[...]
    logits = jnp.dot(z, wfc_ref[...],
                     preferred_element_type=jnp.float32) + bfc_ref[...]
    m = jnp.max(logits, axis=-1, keepdims=True)
    s = logits - m
    o_ref[...] = s - jnp.log(jnp.sum(jnp.exp(s), axis=-1, keepdims=True))


def _head(p0, p1, ba, wb, bb, g, be, wfc, bfc, blk):
    n, d = p0.shape
    c = wfc.shape[1]
    vec = lambda w: pl.BlockSpec((1, w), lambda i: (0, 0))
    mat = lambda a, b: pl.BlockSpec((a, b), lambda i: (0, 0))
    big = lambda: pl.BlockSpec((blk, d), lambda i: (i, 0))
    return pl.pallas_call(
        _head_body,
        grid=(n // blk,),
        in_specs=[big(), big(), vec(d), mat(d, d), vec(d), vec(d),
                  vec(d), mat(d, c), vec(c)],
        out_specs=pl.BlockSpec((blk, c), lambda i: (i, 0)),
        out_shape=jax.ShapeDtypeStruct((n, c), jnp.float32),
    )(p0, p1, ba.reshape(1, d), wb, bb.reshape(1, d), g.reshape(1, d),
      be.reshape(1, d), wfc, bfc.reshape(1, c))


def kernel(x, edge_index, batch, W1a, b1a, W1b, b1b, g1, be1, W2a, b2a, W2b,
           b2b, g2, be2, W3a, b3a, W3b, b3b, g3, be3, Wfc, bfc):
    n, f_in = x.shape
    e = edge_index.shape[1]
    dim = W1a.shape[1]

    n_pad = ((n + 1 + NS * 8 - 1) // (NS * 8)) * (NS * 8)
    ch = (e + NW * CB - 1) // (NW * CB)
    ch = ((ch + NB - 1) // NB) * NB
    e_pad = NW * ch * CB

    src = jnp.pad(edge_index[0], (0, e_pad - e), constant_values=n)
    dst = jnp.pad(edge_index[1], (0, e_pad - e), constant_values=n)
    srcm = src.reshape(NW, ch, CB)
    dstm = dst.reshape(NW, ch, CB)
    zeros = jnp.zeros((n_pad, dim), jnp.float32)

    sc_agg = _make_sc_agg(n_pad, ch, dim)
    blk = n_pad // 8

    t1 = _proj(x, W1a, n_pad, 5)
    p = sc_agg(t1, srcm, dstm, zeros)
    t2 = _mid(p[0], p[1], b1a, W1b, b1b, g1, be1, W2a, blk)
    p = sc_agg(t2, srcm, dstm, zeros)
    t3 = _mid(p[0], p[1], b2a, W2b, b2b, g2, be2, W3a, blk)
    p = sc_agg(t3, srcm, dstm, zeros)
    out = _head(p[0], p[1], b3a, W3b, b3b, g3, be3, Wfc, bfc, blk)
    return out[:n]
